# baseline (speedup 1.0000x reference)
import functools

import jax
import jax.numpy as jnp
from jax import lax
from jax.experimental import pallas as pl
from jax.experimental.pallas import tpu as pltpu
from jax.experimental.pallas import tpu_sc as plsc

N_NODES = 10000
N_EDGES = 320000
D = 128

NC = 2
NS = 16
NW = NC * NS
EDGES_PER_TILE = N_EDGES // NW
CHUNK = 80
N_CHUNKS = EDGES_PER_TILE // CHUNK
N_PAD = 10240
ROWS_PER_SUB = N_PAD // NS


def _sc_gather_scatter(feature, src3, dst3):
    mesh = plsc.VectorSubcoreMesh(core_axis_name="c", subcore_axis_name="s")

    @functools.partial(
        pl.kernel,
        mesh=mesh,
        out_type=jax.ShapeDtypeStruct((NC, N_PAD, D), jnp.float32),
        scratch_types=(
            [pltpu.VMEM((CHUNK,), jnp.int32)] * 4
            + [pltpu.VMEM((CHUNK,), jnp.int32)] * 4
            + [pltpu.VMEM((CHUNK, D), jnp.float32)] * 4
            + [pltpu.VMEM_SHARED((N_PAD, D), jnp.float32)]
            + [pltpu.SemaphoreType.DMA] * 16
        ),
    )
    def k(feature_hbm, src_hbm, dst_hbm, out_hbm,
          src0, src1, src2, src3, dst0, dst1, dst2, dst3,
          rows0, rows1, rows2, rows3, acc,
          sg0, sg1, sg2, sg3, si0, si1, si2, si3,
          sd0, sd1, sd2, sd3, ss0, ss1, ss2, ss3):
        c = lax.axis_index("c")
        s = lax.axis_index("s")
        wid = s * NC + c
        rbase = s * ROWS_PER_SUB
        ebase = wid * EDGES_PER_TILE
        src_i = (src0, src1, src2, src3)
        dst_i = (dst0, dst1, dst2, dst3)
        rows = (rows0, rows1, rows2, rows3)
        sg = (sg0, sg1, sg2, sg3)
        si = (si0, si1, si2, si3)
        sd = (sd0, sd1, sd2, sd3)
        ss = (ss0, ss1, ss2, ss3)

        def zero_row(i, carry):
            for j in range(D // 16):
                rows0[i, pl.ds(j * 16, 16)] = jnp.zeros((16,), jnp.float32)
            return carry

        lax.fori_loop(0, CHUNK, zero_row, 0)
        for t in range(ROWS_PER_SUB // CHUNK):
            pltpu.sync_copy(rows0, acc.at[pl.ds(rbase + t * CHUNK, CHUNK)])
        plsc.subcore_barrier()

        def idx_wait(sl):
            pltpu.make_async_copy(
                src_hbm.at[pl.ds(ebase, CHUNK)], src_i[sl], si[sl]).wait()
            pltpu.make_async_copy(
                dst_hbm.at[pl.ds(ebase, CHUNK)], dst_i[sl], sd[sl]).wait()

        def scat_wait(sl):
            pltpu.make_async_copy(
                rows[sl], acc.at[dst_i[sl]], ss[sl]).wait()

        def step(n, b, first=False):
            b2 = (b + 2) % 4
            b3 = (b + 3) % 4
            bp = (b + 3) % 4
            idx_wait(b2)
            pltpu.async_copy(
                feature_hbm.at[src_i[b2]], rows[b2], sg[b2])
            pltpu.make_async_copy(
                feature_hbm.at[src_i[b]], rows[b], sg[b]).wait()
            if not first:
                scat_wait(bp)
            pltpu.async_copy(
                rows[b], acc.at[dst_i[b]], ss[b], add=True)
            nb = jnp.minimum(n + 3, N_CHUNKS - 1) * CHUNK + ebase
            pltpu.async_copy(
                src_hbm.at[pl.ds(nb, CHUNK)], src_i[b3], si[b3])
            pltpu.async_copy(
                dst_hbm.at[pl.ds(nb, CHUNK)], dst_i[b3], sd[b3])

        pltpu.sync_copy(src_hbm.at[pl.ds(ebase, CHUNK)], src0)
        pltpu.sync_copy(dst_hbm.at[pl.ds(ebase, CHUNK)], dst0)
        pltpu.sync_copy(src_hbm.at[pl.ds(ebase + CHUNK, CHUNK)], src1)
        pltpu.sync_copy(dst_hbm.at[pl.ds(ebase + CHUNK, CHUNK)], dst1)
        pltpu.async_copy(feature_hbm.at[src0], rows0, sg0)
        pltpu.async_copy(feature_hbm.at[src1], rows1, sg1)
        pltpu.async_copy(src_hbm.at[pl.ds(ebase + 2 * CHUNK, CHUNK)], src2, si2)
        pltpu.async_copy(dst_hbm.at[pl.ds(ebase + 2 * CHUNK, CHUNK)], dst2, sd2)

        step(0, 0, first=True)
        step(1, 1)
        step(2, 2)

        def body(g, carry):
            step(4 * g + 3, 3)
            step(4 * g + 4, 0)
            step(4 * g + 5, 1)
            step(4 * g + 6, 2)
            return carry

        lax.fori_loop(0, (N_CHUNKS - 3 - 2) // 4, body, 0)
        nE = N_CHUNKS - 2
        for n in (nE, nE + 1):
            b = n % 4
            pltpu.make_async_copy(
                feature_hbm.at[src_i[b]], rows[b], sg[b]).wait()
            scat_wait((b + 3) % 4)
            pltpu.async_copy(rows[b], acc.at[dst_i[b]], ss[b], add=True)
        scat_wait((nE + 1) % 4)
        idx_wait((nE + 2) % 4)
        plsc.subcore_barrier()

        pltpu.sync_copy(
            acc.at[pl.ds(rbase, ROWS_PER_SUB)],
            out_hbm.at[c, pl.ds(rbase, ROWS_PER_SUB)],
        )

    return k(feature, src3, dst3)


BLK = 1000


def _tc_body(hp_ref, g_ref, be_ref, w_ref, b_ref, o_ref):
    h = hp_ref[0] + hp_ref[1]
    mean = jnp.mean(h, axis=-1, keepdims=True)
    var = jnp.mean((h - mean) ** 2, axis=-1, keepdims=True)
    hn = (h - mean) * lax.rsqrt(var + 1e-5)
    hn = hn * g_ref[...] + be_ref[...]
    o_ref[...] = (
        lax.dot_general(hn, w_ref[...], (((1,), (1,)), ((), ())),
                        preferred_element_type=jnp.float32)
        + b_ref[...]
    )


def _tc_finish(hpart, ln_gamma, ln_beta, W, b):
    grid = N_NODES // BLK
    return pl.pallas_call(
        _tc_body,
        grid=(grid,),
        in_specs=[
            pl.BlockSpec((NC, BLK, D), lambda i: (0, i, 0)),
            pl.BlockSpec((1, D), lambda i: (0, 0)),
            pl.BlockSpec((1, D), lambda i: (0, 0)),
            pl.BlockSpec((D, D), lambda i: (0, 0)),
            pl.BlockSpec((1, D), lambda i: (0, 0)),
        ],
        out_specs=pl.BlockSpec((BLK, D), lambda i: (i, 0)),
        out_shape=jax.ShapeDtypeStruct((N_NODES, D), jnp.float32),
    )(hpart, ln_gamma.reshape(1, D), ln_beta.reshape(1, D), W, b.reshape(1, D))


def kernel(feature, edge_index, ln_gamma, ln_beta, W, b):
    ei = edge_index.astype(jnp.int32)
    hpart = _sc_gather_scatter(feature, ei[0], ei[1])
    return _tc_finish(hpart, ln_gamma, ln_beta, W, b)


# speedup vs baseline: 1.1336x; 1.1336x over previous
import functools

import jax
import jax.numpy as jnp
from jax import lax
from jax.experimental import pallas as pl
from jax.experimental.pallas import tpu as pltpu
from jax.experimental.pallas import tpu_sc as plsc

N_NODES = 10000
N_EDGES = 320000
D = 128

NC = 2
NS = 16
NW = NC * NS
CHUNK = 128
N_BLOCKS = N_EDGES // CHUNK
NB_TILE = N_BLOCKS // NW
ROWS_A = 624
N_STEPS = NB_TILE - 2


def _sc_gather_scatter(feature, edges):
    mesh = plsc.VectorSubcoreMesh(core_axis_name="c", subcore_axis_name="s")

    @functools.partial(
        pl.kernel,
        mesh=mesh,
        out_type=jax.ShapeDtypeStruct((NC, N_NODES, D), jnp.float32),
        scratch_types=(
            [pltpu.VMEM((2, CHUNK), jnp.int32)] * 4
            + [pltpu.VMEM((CHUNK, D), jnp.float32)] * 3
            + [pltpu.VMEM_SHARED((N_NODES, D), jnp.float32)]
            + [pltpu.SemaphoreType.DMA] * 11
        ),
    )
    def k(feature_hbm, edge_hbm, out_hbm,
          ib0, ib1, ib2, ib3, rows0, rows1, rows2, acc,
          sb0, sb1, sb2, sb3, sg0, sg1, sg2, ss0, ss1, ss2, sx):
        c = lax.axis_index("c")
        s = lax.axis_index("s")
        wid = s * NC + c
        rbase = s * ROWS_A
        bbase = wid * NB_TILE
        ib = (ib0, ib1, ib2, ib3)
        rows = (rows0, rows1, rows2)
        sb = (sb0, sb1, sb2, sb3)
        sg = (sg0, sg1, sg2)
        ss = (ss0, ss1, ss2)

        def zero_row(i, carry):
            for j in range(D // 16):
                rows0[i, pl.ds(j * 16, 16)] = jnp.zeros((16,), jnp.float32)
            return carry

        lax.fori_loop(0, CHUNK, zero_row, 0)
        for t in range(ROWS_A // CHUNK):
            pltpu.sync_copy(rows0, acc.at[pl.ds(rbase + t * CHUNK, CHUNK)])
        rem = ROWS_A % CHUNK
        pltpu.sync_copy(
            rows0.at[pl.ds(0, rem)],
            acc.at[pl.ds(rbase + (ROWS_A // CHUNK) * CHUNK, rem)])

        @pl.when(s == 0)
        def _():
            pltpu.sync_copy(
                rows0.at[pl.ds(0, N_NODES - NS * ROWS_A)],
                acc.at[pl.ds(NS * ROWS_A, N_NODES - NS * ROWS_A)])

        plsc.subcore_barrier()

        def eslice(blk):
            return edge_hbm.at[:, pl.ds(blk * CHUNK, CHUNK)]

        def idx_wait(q):
            pltpu.make_async_copy(eslice(0), ib[q], sb[q]).wait()

        def scat_wait(r, q):
            pltpu.make_async_copy(rows[r], acc.at[ib[q].at[1]], ss[r]).wait()

        def step(n, r, q, first=False, last2=False):
            if not first:
                scat_wait((r + 2) % 3, (q + 3) % 4)
            if not last2:
                nxt = jnp.minimum(n + 3, NB_TILE - 1) + bbase
                pltpu.async_copy(eslice(nxt), ib[(q + 3) % 4], sb[(q + 3) % 4])
                idx_wait((q + 2) % 4)
                pltpu.async_copy(
                    feature_hbm.at[ib[(q + 2) % 4].at[0]],
                    rows[(r + 2) % 3], sg[(r + 2) % 3])
            pltpu.make_async_copy(
                feature_hbm.at[ib[q].at[0]], rows[r], sg[r]).wait()
            pltpu.async_copy(rows[r], acc.at[ib[q].at[1]], ss[r], add=True)

        pltpu.sync_copy(eslice(bbase), ib0)
        pltpu.sync_copy(eslice(bbase + 1), ib1)
        pltpu.async_copy(feature_hbm.at[ib0.at[0]], rows0, sg0)
        pltpu.async_copy(feature_hbm.at[ib1.at[0]], rows1, sg1)
        pltpu.async_copy(eslice(bbase + 2), ib2, sb2)

        step(0, 0, 0, first=True)
        step(1, 1, 1)
        step(2, 2, 2)
        step(3, 0, 3)

        def body(g, carry):
            for j in range(12):
                step(12 * g + 4 + j, (4 + j) % 3, (4 + j) % 4)
            return carry

        lax.fori_loop(0, (N_STEPS - 4) // 12, body, 0)
        step(NB_TILE - 2, (NB_TILE - 2) % 3, (NB_TILE - 2) % 4, last2=True)
        step(NB_TILE - 1, (NB_TILE - 1) % 3, (NB_TILE - 1) % 4, last2=True)
        scat_wait((NB_TILE - 1) % 3, (NB_TILE - 1) % 4)
        idx_wait((N_STEPS - 1 + 3) % 4)

        @pl.when(wid < 4)
        def _():
            tb = NW * NB_TILE + wid
            pltpu.sync_copy(eslice(tb), ib0)
            pltpu.async_copy(feature_hbm.at[ib0.at[0]], rows0, sx).wait()
            pltpu.sync_copy(rows0, acc.at[ib0.at[1]], add=True)

        plsc.subcore_barrier()

        pltpu.sync_copy(
            acc.at[pl.ds(rbase, ROWS_A)],
            out_hbm.at[c, pl.ds(rbase, ROWS_A)])

        @pl.when(s == 0)
        def _():
            pltpu.sync_copy(
                acc.at[pl.ds(NS * ROWS_A, N_NODES - NS * ROWS_A)],
                out_hbm.at[c, pl.ds(NS * ROWS_A, N_NODES - NS * ROWS_A)])

    return k(feature, edges)


BLK = 1000


def _tc_body(hp_ref, g_ref, be_ref, w_ref, b_ref, o_ref):
    h = hp_ref[0] + hp_ref[1]
    mean = jnp.mean(h, axis=-1, keepdims=True)
    var = jnp.mean((h - mean) ** 2, axis=-1, keepdims=True)
    hn = (h - mean) * lax.rsqrt(var + 1e-5)
    hn = hn * g_ref[...] + be_ref[...]
    o_ref[...] = (
        lax.dot_general(hn, w_ref[...], (((1,), (1,)), ((), ())),
                        preferred_element_type=jnp.float32)
        + b_ref[...]
    )


def _tc_finish(hpart, ln_gamma, ln_beta, W, b):
    grid = N_NODES // BLK
    return pl.pallas_call(
        _tc_body,
        grid=(grid,),
        in_specs=[
            pl.BlockSpec((NC, BLK, D), lambda i: (0, i, 0)),
            pl.BlockSpec((1, D), lambda i: (0, 0)),
            pl.BlockSpec((1, D), lambda i: (0, 0)),
            pl.BlockSpec((D, D), lambda i: (0, 0)),
            pl.BlockSpec((1, D), lambda i: (0, 0)),
        ],
        out_specs=pl.BlockSpec((BLK, D), lambda i: (i, 0)),
        out_shape=jax.ShapeDtypeStruct((N_NODES, D), jnp.float32),
    )(hpart, ln_gamma.reshape(1, D), ln_beta.reshape(1, D), W, b.reshape(1, D))


def kernel(feature, edge_index, ln_gamma, ln_beta, W, b):
    ei = edge_index.astype(jnp.int32)
    hpart = _sc_gather_scatter(feature, ei)
    return _tc_finish(hpart, ln_gamma, ln_beta, W, b)


# speedup vs baseline: 1.1602x; 1.0235x over previous
import functools

import jax
import jax.numpy as jnp
from jax import lax
from jax.experimental import pallas as pl
from jax.experimental.pallas import tpu as pltpu
from jax.experimental.pallas import tpu_sc as plsc

N_NODES = 10000
N_EDGES = 320000
D = 128

NC = 2
NS = 16
NW = NC * NS
CHUNK = 128
N_BLOCKS = N_EDGES // CHUNK
NB_TILE = N_BLOCKS // NW
ROWS_A = 624
N_STEPS = NB_TILE - 2


def _sc_gather_scatter(feature, edges):
    mesh = plsc.VectorSubcoreMesh(core_axis_name="c", subcore_axis_name="s")

    @functools.partial(
        pl.kernel,
        mesh=mesh,
        out_type=jax.ShapeDtypeStruct((NC, N_NODES, D), jnp.float32),
        scratch_types=(
            [pltpu.VMEM((2, CHUNK), jnp.int32)] * 4
            + [pltpu.VMEM((CHUNK, D), jnp.float32)] * 3
            + [pltpu.VMEM_SHARED((N_NODES, D), jnp.float32)]
            + [pltpu.SemaphoreType.DMA] * 11
        ),
    )
    def k(feature_hbm, edge_hbm, out_hbm,
          ib0, ib1, ib2, ib3, rows0, rows1, rows2, acc,
          sb0, sb1, sb2, sb3, sg0, sg1, sg2, ss0, ss1, ss2, sx):
        c = lax.axis_index("c")
        s = lax.axis_index("s")
        wid = s * NC + c
        rbase = s * ROWS_A
        bbase = wid * NB_TILE
        ib = (ib0, ib1, ib2, ib3)
        rows = (rows0, rows1, rows2)
        sb = (sb0, sb1, sb2, sb3)
        sg = (sg0, sg1, sg2)
        ss = (ss0, ss1, ss2)

        def zero_row(i, carry):
            for j in range(D // 16):
                rows0[i, pl.ds(j * 16, 16)] = jnp.zeros((16,), jnp.float32)
            return carry

        lax.fori_loop(0, CHUNK, zero_row, 0)
        for t in range(ROWS_A // CHUNK):
            pltpu.sync_copy(rows0, acc.at[pl.ds(rbase + t * CHUNK, CHUNK)])
        rem = ROWS_A % CHUNK
        pltpu.sync_copy(
            rows0.at[pl.ds(0, rem)],
            acc.at[pl.ds(rbase + (ROWS_A // CHUNK) * CHUNK, rem)])

        @pl.when(s == 0)
        def _():
            pltpu.sync_copy(
                rows0.at[pl.ds(0, N_NODES - NS * ROWS_A)],
                acc.at[pl.ds(NS * ROWS_A, N_NODES - NS * ROWS_A)])

        plsc.subcore_barrier()

        def eslice(blk):
            return edge_hbm.at[:, pl.ds(blk * CHUNK, CHUNK)]

        def idx_wait(q):
            pltpu.make_async_copy(eslice(0), ib[q], sb[q]).wait()

        def scat_wait(r, q):
            pltpu.make_async_copy(rows[r], acc.at[ib[q].at[1]], ss[r]).wait()

        def step(n, r, q, first=False, last2=False):
            if not first:
                scat_wait((r + 2) % 3, (q + 3) % 4)
            if not last2:
                nxt = jnp.minimum(n + 3, NB_TILE - 1) + bbase
                pltpu.async_copy(eslice(nxt), ib[(q + 3) % 4], sb[(q + 3) % 4])
                idx_wait((q + 2) % 4)
                pltpu.async_copy(
                    feature_hbm.at[ib[(q + 2) % 4].at[0]],
                    rows[(r + 2) % 3], sg[(r + 2) % 3])
            pltpu.make_async_copy(
                feature_hbm.at[ib[q].at[0]], rows[r], sg[r]).wait()
            pltpu.async_copy(rows[r], acc.at[ib[q].at[1]], ss[r], add=True)

        pltpu.sync_copy(eslice(bbase), ib0)
        pltpu.sync_copy(eslice(bbase + 1), ib1)
        pltpu.async_copy(feature_hbm.at[ib0.at[0]], rows0, sg0)
        pltpu.async_copy(feature_hbm.at[ib1.at[0]], rows1, sg1)
        pltpu.async_copy(eslice(bbase + 2), ib2, sb2)

        step(0, 0, 0, first=True)
        step(1, 1, 1)
        step(2, 2, 2)
        step(3, 0, 3)

        def body(g, carry):
            for j in range(12):
                step(12 * g + 4 + j, (4 + j) % 3, (4 + j) % 4)
            return carry

        lax.fori_loop(0, (N_STEPS - 4) // 12, body, 0)
        step(NB_TILE - 2, (NB_TILE - 2) % 3, (NB_TILE - 2) % 4, last2=True)
        step(NB_TILE - 1, (NB_TILE - 1) % 3, (NB_TILE - 1) % 4, last2=True)
        scat_wait((NB_TILE - 1) % 3, (NB_TILE - 1) % 4)
        idx_wait((N_STEPS - 1 + 3) % 4)

        @pl.when(wid < 4)
        def _():
            tb = NW * NB_TILE + wid
            pltpu.sync_copy(eslice(tb), ib0)
            pltpu.async_copy(feature_hbm.at[ib0.at[0]], rows0, sx).wait()
            pltpu.sync_copy(rows0, acc.at[ib0.at[1]], add=True)

        plsc.subcore_barrier()

        pltpu.sync_copy(
            acc.at[pl.ds(rbase, ROWS_A)],
            out_hbm.at[c, pl.ds(rbase, ROWS_A)])

        @pl.when(s == 0)
        def _():
            pltpu.sync_copy(
                acc.at[pl.ds(NS * ROWS_A, N_NODES - NS * ROWS_A)],
                out_hbm.at[c, pl.ds(NS * ROWS_A, N_NODES - NS * ROWS_A)])

    return k(feature, edges)


BLK = 2000


def _tc_body(hp_ref, g_ref, be_ref, w_ref, b_ref, o_ref):
    h = hp_ref[0] + hp_ref[1]
    mean = jnp.mean(h, axis=-1, keepdims=True)
    var = jnp.mean((h - mean) ** 2, axis=-1, keepdims=True)
    hn = (h - mean) * lax.rsqrt(var + 1e-5)
    hn = hn * g_ref[...] + be_ref[...]
    o_ref[...] = (
        lax.dot_general(hn, w_ref[...], (((1,), (1,)), ((), ())),
                        preferred_element_type=jnp.float32)
        + b_ref[...]
    )


def _tc_finish(hpart, ln_gamma, ln_beta, W, b):
    grid = N_NODES // BLK
    return pl.pallas_call(
        _tc_body,
        grid=(grid,),
        in_specs=[
            pl.BlockSpec((NC, BLK, D), lambda i: (0, i, 0)),
            pl.BlockSpec((1, D), lambda i: (0, 0)),
            pl.BlockSpec((1, D), lambda i: (0, 0)),
            pl.BlockSpec((D, D), lambda i: (0, 0)),
            pl.BlockSpec((1, D), lambda i: (0, 0)),
        ],
        out_specs=pl.BlockSpec((BLK, D), lambda i: (i, 0)),
        out_shape=jax.ShapeDtypeStruct((N_NODES, D), jnp.float32),
    )(hpart, ln_gamma.reshape(1, D), ln_beta.reshape(1, D), W, b.reshape(1, D))


def kernel(feature, edge_index, ln_gamma, ln_beta, W, b):
    ei = edge_index.astype(jnp.int32)
    hpart = _sc_gather_scatter(feature, ei)
    return _tc_finish(hpart, ln_gamma, ln_beta, W, b)
